# Initial kernel scaffold; baseline (speedup 1.0000x reference)
import jax
import jax.numpy as jnp
from jax.experimental import pallas as pl


def kernel(x, edge_index, batch, batch_size, edge_attr, W1, b1, W2, b2, W3, b3, Ws, bs, Wg, bg, Wl2, bl2, Wl3, bl3):
    raise NotImplementedError("write your pallas kernel here")


# speedup vs baseline: 22.4097x; 22.4097x over previous
import functools

import jax
import jax.numpy as jnp
from jax import lax
from jax.experimental import pallas as pl
from jax.experimental.pallas import tpu as pltpu
from jax.experimental.pallas import tpu_sc as plsc

N = 10000
E = 320000
B = 8
NPER = 1250
NH = 64
K = 625

NTILES = 32
EPT = E // NTILES
CH = 125
NCH = EPT // CH
ZROWS = 1000

_MESH = plsc.VectorSubcoreMesh(core_axis_name="c", subcore_axis_name="s")


def _sc_degree_body(c3, zeros_hbm, ones_hbm, out, cidx, ones_v, deg_sh):
    cid = lax.axis_index("c")
    sid = lax.axis_index("s")
    wid = cid * 16 + sid
    pltpu.sync_copy(c3.at[wid], cidx)
    pltpu.sync_copy(ones_hbm, ones_v)

    @pl.when(sid < N // ZROWS)
    def _():
        pltpu.sync_copy(zeros_hbm, deg_sh.at[pl.ds(sid * ZROWS, ZROWS)])

    plsc.subcore_barrier()

    def body(j, _):
        pltpu.sync_copy(ones_v, deg_sh.at[cidx.at[j]], add=True)
        return _

    lax.fori_loop(0, NCH, body, None)
    plsc.subcore_barrier()

    @pl.when(sid < N // ZROWS)
    def _():
        pltpu.sync_copy(deg_sh.at[pl.ds(sid * ZROWS, ZROWS)],
                        out.at[cid, pl.ds(sid * ZROWS, ZROWS)])


_W1 = 16

_sc_degree = pl.kernel(
    _sc_degree_body,
    out_type=jax.ShapeDtypeStruct((2, N, _W1), jnp.float32),
    mesh=_MESH,
    compiler_params=pltpu.CompilerParams(use_tc_tiling_on_sc=False),
    scratch_types=[
        pltpu.VMEM((NCH, CH), jnp.int32),
        pltpu.VMEM((CH, _W1), jnp.float32),
        pltpu.VMEM_SHARED((N, _W1), jnp.float32),
    ],
)


def _sc_scatter_body(u, r3, c3, zeros_hbm, out, ridx, cidx, rows, s_sh):
    cid = lax.axis_index("c")
    sid = lax.axis_index("s")
    wid = cid * 16 + sid
    pltpu.sync_copy(r3.at[wid], ridx)
    pltpu.sync_copy(c3.at[wid], cidx)

    @pl.when(sid < N // ZROWS)
    def _():
        pltpu.sync_copy(zeros_hbm, s_sh.at[pl.ds(sid * ZROWS, ZROWS)])

    plsc.subcore_barrier()

    def body(j, _):
        pltpu.sync_copy(u.at[ridx.at[j]], rows)
        pltpu.sync_copy(rows, s_sh.at[cidx.at[j]], add=True)
        return _

    lax.fori_loop(0, NCH, body, None)
    plsc.subcore_barrier()

    @pl.when(sid < N // ZROWS)
    def _():
        pltpu.sync_copy(s_sh.at[pl.ds(sid * ZROWS, ZROWS)],
                        out.at[cid, pl.ds(sid * ZROWS, ZROWS)])


def _make_sc_scatter(w):
    return pl.kernel(
        _sc_scatter_body,
        out_type=jax.ShapeDtypeStruct((2, N, w), jnp.float32),
        mesh=_MESH,
        compiler_params=pltpu.CompilerParams(use_tc_tiling_on_sc=False),
        scratch_types=[
            pltpu.VMEM((NCH, CH), jnp.int32),
            pltpu.VMEM((NCH, CH), jnp.int32),
            pltpu.VMEM((CH, w), jnp.float32),
            pltpu.VMEM_SHARED((N, w), jnp.float32),
        ],
    )


_sc_scatter64 = _make_sc_scatter(NH)
_sc_scatter16 = _make_sc_scatter(_W1)


_R = 1000


def _tc1_body(degp, x, w1, dinv, u1):
    deg = degp[0] + degp[1] + 1.0
    di = lax.rsqrt(deg)
    dinv[...] = di
    u1[...] = jnp.dot(x[...], w1[...], preferred_element_type=jnp.float32) * di


def _tc_layer_body(sp, uprev, dinv, b, wnext, hprev, unext):
    h = jnp.maximum(dinv[...] * (sp[0] + sp[1] + uprev[...]) + b[...], 0.0)
    hprev[...] = h
    unext[...] = jnp.dot(h, wnext[...], preferred_element_type=jnp.float32) * dinv[...]


def _tc_score_body(sp, u3, dinv, b3, h1, h2, wsa, wsb, wsc, h3, us):
    h = jnp.maximum(dinv[...] * (sp[0] + sp[1] + u3[...]) + b3[...], 0.0)
    h3[...] = h
    s = (jnp.dot(h1[...], wsa[...], preferred_element_type=jnp.float32)
         + jnp.dot(h2[...], wsb[...], preferred_element_type=jnp.float32)
         + jnp.dot(h, wsc[...], preferred_element_type=jnp.float32))
    us[...] = jnp.broadcast_to(s * dinv[...], (s.shape[0], _W1))


def _full_spec(shape):
    return pl.BlockSpec(shape, lambda i: tuple(0 for _ in shape))


_tc1 = pl.pallas_call(
    _tc1_body,
    grid=(N // _R,),
    in_specs=[pl.BlockSpec((2, _R, 1), lambda i: (0, i, 0)),
              pl.BlockSpec((_R, 128), lambda i: (i, 0)),
              _full_spec((128, NH))],
    out_specs=[pl.BlockSpec((_R, 1), lambda i: (i, 0)),
               pl.BlockSpec((_R, NH), lambda i: (i, 0))],
    out_shape=[jax.ShapeDtypeStruct((N, 1), jnp.float32),
               jax.ShapeDtypeStruct((N, NH), jnp.float32)],
)

_tc_layer = pl.pallas_call(
    _tc_layer_body,
    grid=(N // _R,),
    in_specs=[pl.BlockSpec((2, _R, NH), lambda i: (0, i, 0)),
              pl.BlockSpec((_R, NH), lambda i: (i, 0)),
              pl.BlockSpec((_R, 1), lambda i: (i, 0)),
              _full_spec((1, NH)),
              _full_spec((NH, NH))],
    out_specs=[pl.BlockSpec((_R, NH), lambda i: (i, 0)),
               pl.BlockSpec((_R, NH), lambda i: (i, 0))],
    out_shape=[jax.ShapeDtypeStruct((N, NH), jnp.float32),
               jax.ShapeDtypeStruct((N, NH), jnp.float32)],
)

_tc_score = pl.pallas_call(
    _tc_score_body,
    grid=(N // _R,),
    in_specs=[pl.BlockSpec((2, _R, NH), lambda i: (0, i, 0)),
              pl.BlockSpec((_R, NH), lambda i: (i, 0)),
              pl.BlockSpec((_R, 1), lambda i: (i, 0)),
              _full_spec((1, NH)),
              pl.BlockSpec((_R, NH), lambda i: (i, 0)),
              pl.BlockSpec((_R, NH), lambda i: (i, 0)),
              _full_spec((NH, 1)),
              _full_spec((NH, 1)),
              _full_spec((NH, 1))],
    out_specs=[pl.BlockSpec((_R, NH), lambda i: (i, 0)),
               pl.BlockSpec((_R, _W1), lambda i: (i, 0))],
    out_shape=[jax.ShapeDtypeStruct((N, NH), jnp.float32),
               jax.ShapeDtypeStruct((N, _W1), jnp.float32)],
)


def _tc_mask_body(ssp, us, dinv, bs, gsel, negadd):
    score = dinv[...] * (ssp[0] + ssp[1] + us[...]) + bs[...]

    bits = lax.bitcast_convert_type(score, jnp.uint32)
    top = jnp.uint32(0x80000000)
    ukey = bits ^ jnp.where(bits >= top, jnp.uint32(0xFFFFFFFF), top)

    t = jnp.zeros((B, 1), jnp.uint32)
    for b in range(31, -1, -1):
        cand = t | jnp.uint32(1 << b)
        cnt = jnp.sum((ukey >= cand).astype(jnp.int32), axis=1, keepdims=True)
        t = jnp.where(cnt >= K, cand, t)

    gt = ukey > t
    tie = ukey == t
    r = K - jnp.sum(gt.astype(jnp.int32), axis=1, keepdims=True)
    idx = lax.broadcasted_iota(jnp.int32, (B, NPER), 1)
    lo = jnp.zeros((B, 1), jnp.int32)
    hi = jnp.full((B, 1), NPER - 1, jnp.int32)
    for _ in range(11):
        mid = (lo + hi) // 2
        cnt = jnp.sum((tie & (idx <= mid)).astype(jnp.int32), axis=1,
                      keepdims=True)
        ok = cnt >= r
        hi = jnp.where(ok, mid, hi)
        lo = jnp.where(ok, lo, mid + 1)
    mask = gt | (tie & (idx <= lo))

    gsel[...] = jnp.where(mask, jnp.tanh(score), 0.0)
    negadd[...] = jnp.where(mask, 0.0, jnp.float32(-3.4e38))


_tc_mask = pl.pallas_call(
    _tc_mask_body,
    out_shape=[jax.ShapeDtypeStruct((B, NPER), jnp.float32),
               jax.ShapeDtypeStruct((B, NPER), jnp.float32)],
)


def _tc_pool_body(h1, h2, h3, gsel, neg, o1, o2, o3, o4, o5, o6):
    g2 = gsel[0]
    n2 = neg[0]
    for h, om, os in ((h1, o1, o4), (h2, o2, o5), (h3, o3, o6)):
        v = h[0] * g2
        om[0] = jnp.max(v + n2, axis=0, keepdims=True)
        os[0] = jnp.sum(v, axis=0, keepdims=True) * (1.0 / K)


_tc_pool = pl.pallas_call(
    _tc_pool_body,
    grid=(B,),
    in_specs=[pl.BlockSpec((1, NPER, NH), lambda g: (g, 0, 0)),
              pl.BlockSpec((1, NPER, NH), lambda g: (g, 0, 0)),
              pl.BlockSpec((1, NPER, NH), lambda g: (g, 0, 0)),
              pl.BlockSpec((1, NPER, 1), lambda g: (g, 0, 0)),
              pl.BlockSpec((1, NPER, 1), lambda g: (g, 0, 0))],
    out_specs=[pl.BlockSpec((1, 1, NH), lambda g: (g, 0, 0))] * 6,
    out_shape=[jax.ShapeDtypeStruct((B, 1, NH), jnp.float32)] * 6,
)


def _tc_mlp_body(xg, wg, bg, wl2, bl2, wl3, bl3, out):
    y = jnp.maximum(jnp.dot(xg[...], wg[...], preferred_element_type=jnp.float32)
                    + bg[...], 0.0)
    y = jnp.maximum(jnp.dot(y, wl2[...], preferred_element_type=jnp.float32)
                    + bl2[...], 0.0)
    y = jnp.dot(y, wl3[...], preferred_element_type=jnp.float32) + bl3[...]
    z = y - jnp.max(y, axis=1, keepdims=True)
    out[...] = z - jnp.log(jnp.sum(jnp.exp(z), axis=1, keepdims=True))


_tc_mlp = pl.pallas_call(
    _tc_mlp_body,
    out_shape=jax.ShapeDtypeStruct((B, 10), jnp.float32),
)


def kernel(x, edge_index, batch, batch_size, edge_attr, W1, b1, W2, b2, W3, b3,
           Ws, bs, Wg, bg, Wl2, bl2, Wl3, bl3):
    del batch, batch_size, edge_attr
    r3 = edge_index[0].astype(jnp.int32).reshape(NTILES, NCH, CH)
    c3 = edge_index[1].astype(jnp.int32).reshape(NTILES, NCH, CH)
    z64 = jnp.zeros((ZROWS, NH), jnp.float32)
    z16 = jnp.zeros((ZROWS, _W1), jnp.float32)
    ones16 = jnp.ones((CH, _W1), jnp.float32)

    degp = _sc_degree(c3, z16, ones16)
    dinv, u1 = _tc1(degp[:, :, 0:1], x, W1)

    s1 = _sc_scatter64(u1, r3, c3, z64)
    h1, u2 = _tc_layer(s1, u1, dinv, b1.reshape(1, NH), W2)

    s2 = _sc_scatter64(u2, r3, c3, z64)
    h2, u3 = _tc_layer(s2, u2, dinv, b2.reshape(1, NH), W3)

    s3 = _sc_scatter64(u3, r3, c3, z64)
    h3, us = _tc_score(s3, u3, dinv, b3.reshape(1, NH), h1, h2,
                       Ws[0:NH], Ws[NH:2 * NH], Ws[2 * NH:3 * NH])

    ss = _sc_scatter16(us, r3, c3, z16)

    gsel, negadd = _tc_mask(ss[:, :, 0].reshape(2, B, NPER),
                            us[:, 0].reshape(B, NPER),
                            dinv.reshape(B, NPER),
                            bs.reshape(1, 1))
    gm1, gm2, gm3, ga1, ga2, ga3 = _tc_pool(
        h1.reshape(B, NPER, NH), h2.reshape(B, NPER, NH),
        h3.reshape(B, NPER, NH),
        gsel.reshape(B, NPER, 1), negadd.reshape(B, NPER, 1))
    xg = jnp.concatenate([gm1, gm2, gm3, ga1, ga2, ga3], axis=2).reshape(B, 6 * NH)
    return _tc_mlp(xg, Wg, bg.reshape(1, NH),
                   Wl2, bl2.reshape(1, NH // 2),
                   Wl3, bl3.reshape(1, 10))


# speedup vs baseline: 36.0744x; 1.6098x over previous
import functools

import jax
import jax.numpy as jnp
from jax import lax
from jax.experimental import pallas as pl
from jax.experimental.pallas import tpu as pltpu
from jax.experimental.pallas import tpu_sc as plsc

N = 10000
E = 320000
B = 8
NPER = 1250
NH = 64
K = 625

NTILES = 32
EPT = E // NTILES
CH = 125
NCH = EPT // CH
ZROWS = 1000

_MESH = plsc.VectorSubcoreMesh(core_axis_name="c", subcore_axis_name="s")


def _sc_degree_body(c3, zeros_hbm, ones_hbm, out, cidx, ones_v, deg_sh):
    cid = lax.axis_index("c")
    sid = lax.axis_index("s")
    wid = cid * 16 + sid
    pltpu.sync_copy(c3.at[wid], cidx)
    pltpu.sync_copy(ones_hbm, ones_v)

    @pl.when(sid < N // ZROWS)
    def _():
        pltpu.sync_copy(zeros_hbm, deg_sh.at[pl.ds(sid * ZROWS, ZROWS)])

    plsc.subcore_barrier()

    def body(j, _):
        pltpu.sync_copy(ones_v, deg_sh.at[cidx.at[j]], add=True)
        return _

    lax.fori_loop(0, NCH, body, None)
    plsc.subcore_barrier()

    @pl.when(sid < N // ZROWS)
    def _():
        pltpu.sync_copy(deg_sh.at[pl.ds(sid * ZROWS, ZROWS)],
                        out.at[cid, pl.ds(sid * ZROWS, ZROWS)])


_W1 = 16

_sc_degree = pl.kernel(
    _sc_degree_body,
    out_type=jax.ShapeDtypeStruct((2, N, _W1), jnp.float32),
    mesh=_MESH,
    compiler_params=pltpu.CompilerParams(use_tc_tiling_on_sc=False),
    scratch_types=[
        pltpu.VMEM((NCH, CH), jnp.int32),
        pltpu.VMEM((CH, _W1), jnp.float32),
        pltpu.VMEM_SHARED((N, _W1), jnp.float32),
    ],
)


_D = 4


def _sc_scatter_body(u, r3, c3, zeros_hbm, out, ridx, cidx, rows, gsem, ssem,
                     s_sh):
    cid = lax.axis_index("c")
    sid = lax.axis_index("s")
    wid = cid * 16 + sid
    pltpu.sync_copy(r3.at[wid], ridx)
    pltpu.sync_copy(c3.at[wid], cidx)

    @pl.when(sid < N // ZROWS)
    def _():
        pltpu.sync_copy(zeros_hbm, s_sh.at[pl.ds(sid * ZROWS, ZROWS)])

    plsc.subcore_barrier()

    for j in range(_D):
        pltpu.async_copy(u.at[ridx.at[j]], rows.at[j], gsem.at[j])

    def body(j, _):
        b = lax.rem(j, _D)
        pltpu.make_async_copy(u.at[ridx.at[j]], rows.at[b], gsem.at[b]).wait()
        pltpu.async_copy(rows.at[b], s_sh.at[cidx.at[j]], ssem.at[b], add=True)

        @pl.when(j + _D < NCH)
        def _():
            pltpu.make_async_copy(rows.at[b], s_sh.at[cidx.at[j]],
                                  ssem.at[b]).wait()
            pltpu.async_copy(u.at[ridx.at[j + _D]], rows.at[b], gsem.at[b])
        return _

    lax.fori_loop(0, NCH, body, None)
    for j in range(NCH - _D, NCH):
        b = j % _D
        pltpu.make_async_copy(rows.at[b], s_sh.at[cidx.at[j]],
                              ssem.at[b]).wait()
    plsc.subcore_barrier()

    @pl.when(sid < N // ZROWS)
    def _():
        pltpu.sync_copy(s_sh.at[pl.ds(sid * ZROWS, ZROWS)],
                        out.at[cid, pl.ds(sid * ZROWS, ZROWS)])


def _make_sc_scatter(w):
    return pl.kernel(
        _sc_scatter_body,
        out_type=jax.ShapeDtypeStruct((2, N, w), jnp.float32),
        mesh=_MESH,
        compiler_params=pltpu.CompilerParams(use_tc_tiling_on_sc=False),
        scratch_types=[
            pltpu.VMEM((NCH, CH), jnp.int32),
            pltpu.VMEM((NCH, CH), jnp.int32),
            pltpu.VMEM((_D, CH, w), jnp.float32),
            pltpu.SemaphoreType.DMA((_D,)),
            pltpu.SemaphoreType.DMA((_D,)),
            pltpu.VMEM_SHARED((N, w), jnp.float32),
        ],
    )


_sc_scatter64 = _make_sc_scatter(NH)
_sc_scatter16 = _make_sc_scatter(_W1)


_R = 1000


def _tc1_body(degp, x, w1, dinv, u1):
    deg = degp[0] + degp[1] + 1.0
    di = lax.rsqrt(deg)
    dinv[...] = di
    u1[...] = jnp.dot(x[...], w1[...], preferred_element_type=jnp.float32) * di


def _tc_layer_body(sp, uprev, dinv, b, wnext, hprev, unext):
    h = jnp.maximum(dinv[...] * (sp[0] + sp[1] + uprev[...]) + b[...], 0.0)
    hprev[...] = h
    unext[...] = jnp.dot(h, wnext[...], preferred_element_type=jnp.float32) * dinv[...]


def _tc_score_body(sp, u3, dinv, b3, h1, h2, wsa, wsb, wsc, h3, us):
    h = jnp.maximum(dinv[...] * (sp[0] + sp[1] + u3[...]) + b3[...], 0.0)
    h3[...] = h
    s = (jnp.dot(h1[...], wsa[...], preferred_element_type=jnp.float32)
         + jnp.dot(h2[...], wsb[...], preferred_element_type=jnp.float32)
         + jnp.dot(h, wsc[...], preferred_element_type=jnp.float32))
    us[...] = jnp.broadcast_to(s * dinv[...], (s.shape[0], _W1))


def _full_spec(shape):
    return pl.BlockSpec(shape, lambda i: tuple(0 for _ in shape))


_tc1 = pl.pallas_call(
    _tc1_body,
    grid=(N // _R,),
    in_specs=[pl.BlockSpec((2, _R, 1), lambda i: (0, i, 0)),
              pl.BlockSpec((_R, 128), lambda i: (i, 0)),
              _full_spec((128, NH))],
    out_specs=[pl.BlockSpec((_R, 1), lambda i: (i, 0)),
               pl.BlockSpec((_R, NH), lambda i: (i, 0))],
    out_shape=[jax.ShapeDtypeStruct((N, 1), jnp.float32),
               jax.ShapeDtypeStruct((N, NH), jnp.float32)],
)

_tc_layer = pl.pallas_call(
    _tc_layer_body,
    grid=(N // _R,),
    in_specs=[pl.BlockSpec((2, _R, NH), lambda i: (0, i, 0)),
              pl.BlockSpec((_R, NH), lambda i: (i, 0)),
              pl.BlockSpec((_R, 1), lambda i: (i, 0)),
              _full_spec((1, NH)),
              _full_spec((NH, NH))],
    out_specs=[pl.BlockSpec((_R, NH), lambda i: (i, 0)),
               pl.BlockSpec((_R, NH), lambda i: (i, 0))],
    out_shape=[jax.ShapeDtypeStruct((N, NH), jnp.float32),
               jax.ShapeDtypeStruct((N, NH), jnp.float32)],
)

_tc_score = pl.pallas_call(
    _tc_score_body,
    grid=(N // _R,),
    in_specs=[pl.BlockSpec((2, _R, NH), lambda i: (0, i, 0)),
              pl.BlockSpec((_R, NH), lambda i: (i, 0)),
              pl.BlockSpec((_R, 1), lambda i: (i, 0)),
              _full_spec((1, NH)),
              pl.BlockSpec((_R, NH), lambda i: (i, 0)),
              pl.BlockSpec((_R, NH), lambda i: (i, 0)),
              _full_spec((NH, 1)),
              _full_spec((NH, 1)),
              _full_spec((NH, 1))],
    out_specs=[pl.BlockSpec((_R, NH), lambda i: (i, 0)),
               pl.BlockSpec((_R, _W1), lambda i: (i, 0))],
    out_shape=[jax.ShapeDtypeStruct((N, NH), jnp.float32),
               jax.ShapeDtypeStruct((N, _W1), jnp.float32)],
)


def _tc_mask_body(ssp, us, dinv, bs, gsel, negadd):
    score = dinv[...] * (ssp[0] + ssp[1] + us[...]) + bs[...]

    bits = lax.bitcast_convert_type(score, jnp.uint32)
    top = jnp.uint32(0x80000000)
    ukey = bits ^ jnp.where(bits >= top, jnp.uint32(0xFFFFFFFF), top)

    t = jnp.zeros((B, 1), jnp.uint32)
    for b in range(31, -1, -1):
        cand = t | jnp.uint32(1 << b)
        cnt = jnp.sum((ukey >= cand).astype(jnp.int32), axis=1, keepdims=True)
        t = jnp.where(cnt >= K, cand, t)

    gt = ukey > t
    tie = ukey == t
    r = K - jnp.sum(gt.astype(jnp.int32), axis=1, keepdims=True)
    idx = lax.broadcasted_iota(jnp.int32, (B, NPER), 1)
    lo = jnp.zeros((B, 1), jnp.int32)
    hi = jnp.full((B, 1), NPER - 1, jnp.int32)
    for _ in range(11):
        mid = (lo + hi) // 2
        cnt = jnp.sum((tie & (idx <= mid)).astype(jnp.int32), axis=1,
                      keepdims=True)
        ok = cnt >= r
        hi = jnp.where(ok, mid, hi)
        lo = jnp.where(ok, lo, mid + 1)
    mask = gt | (tie & (idx <= lo))

    gsel[...] = jnp.where(mask, jnp.tanh(score), 0.0)
    negadd[...] = jnp.where(mask, 0.0, jnp.float32(-3.4e38))


_tc_mask = pl.pallas_call(
    _tc_mask_body,
    out_shape=[jax.ShapeDtypeStruct((B, NPER), jnp.float32),
               jax.ShapeDtypeStruct((B, NPER), jnp.float32)],
)


def _tc_pool_body(h1, h2, h3, gsel, neg, o1, o2, o3, o4, o5, o6):
    g2 = gsel[0]
    n2 = neg[0]
    for h, om, os in ((h1, o1, o4), (h2, o2, o5), (h3, o3, o6)):
        v = h[0] * g2
        om[0] = jnp.max(v + n2, axis=0, keepdims=True)
        os[0] = jnp.sum(v, axis=0, keepdims=True) * (1.0 / K)


_tc_pool = pl.pallas_call(
    _tc_pool_body,
    grid=(B,),
    in_specs=[pl.BlockSpec((1, NPER, NH), lambda g: (g, 0, 0)),
              pl.BlockSpec((1, NPER, NH), lambda g: (g, 0, 0)),
              pl.BlockSpec((1, NPER, NH), lambda g: (g, 0, 0)),
              pl.BlockSpec((1, NPER, 1), lambda g: (g, 0, 0)),
              pl.BlockSpec((1, NPER, 1), lambda g: (g, 0, 0))],
    out_specs=[pl.BlockSpec((1, 1, NH), lambda g: (g, 0, 0))] * 6,
    out_shape=[jax.ShapeDtypeStruct((B, 1, NH), jnp.float32)] * 6,
)


def _tc_mlp_body(xg, wg, bg, wl2, bl2, wl3, bl3, out):
    y = jnp.maximum(jnp.dot(xg[...], wg[...], preferred_element_type=jnp.float32)
                    + bg[...], 0.0)
    y = jnp.maximum(jnp.dot(y, wl2[...], preferred_element_type=jnp.float32)
                    + bl2[...], 0.0)
    y = jnp.dot(y, wl3[...], preferred_element_type=jnp.float32) + bl3[...]
    z = y - jnp.max(y, axis=1, keepdims=True)
    out[...] = z - jnp.log(jnp.sum(jnp.exp(z), axis=1, keepdims=True))


_tc_mlp = pl.pallas_call(
    _tc_mlp_body,
    out_shape=jax.ShapeDtypeStruct((B, 10), jnp.float32),
)


def kernel(x, edge_index, batch, batch_size, edge_attr, W1, b1, W2, b2, W3, b3,
           Ws, bs, Wg, bg, Wl2, bl2, Wl3, bl3):
    del batch, batch_size, edge_attr
    r3 = edge_index[0].astype(jnp.int32).reshape(NTILES, NCH, CH)
    c3 = edge_index[1].astype(jnp.int32).reshape(NTILES, NCH, CH)
    z64 = jnp.zeros((ZROWS, NH), jnp.float32)
    z16 = jnp.zeros((ZROWS, _W1), jnp.float32)
    ones16 = jnp.ones((CH, _W1), jnp.float32)

    degp = _sc_degree(c3, z16, ones16)
    dinv, u1 = _tc1(degp[:, :, 0:1], x, W1)

    s1 = _sc_scatter64(u1, r3, c3, z64)
    h1, u2 = _tc_layer(s1, u1, dinv, b1.reshape(1, NH), W2)

    s2 = _sc_scatter64(u2, r3, c3, z64)
    h2, u3 = _tc_layer(s2, u2, dinv, b2.reshape(1, NH), W3)

    s3 = _sc_scatter64(u3, r3, c3, z64)
    h3, us = _tc_score(s3, u3, dinv, b3.reshape(1, NH), h1, h2,
                       Ws[0:NH], Ws[NH:2 * NH], Ws[2 * NH:3 * NH])

    ss = _sc_scatter16(us, r3, c3, z16)

    gsel, negadd = _tc_mask(ss[:, :, 0].reshape(2, B, NPER),
                            us[:, 0].reshape(B, NPER),
                            dinv.reshape(B, NPER),
                            bs.reshape(1, 1))
    gm1, gm2, gm3, ga1, ga2, ga3 = _tc_pool(
        h1.reshape(B, NPER, NH), h2.reshape(B, NPER, NH),
        h3.reshape(B, NPER, NH),
        gsel.reshape(B, NPER, 1), negadd.reshape(B, NPER, 1))
    xg = jnp.concatenate([gm1, gm2, gm3, ga1, ga2, ga3], axis=2).reshape(B, 6 * NH)
    return _tc_mlp(xg, Wg, bg.reshape(1, NH),
                   Wl2, bl2.reshape(1, NH // 2),
                   Wl3, bl3.reshape(1, 10))


# speedup vs baseline: 36.4795x; 1.0112x over previous
import functools

import jax
import jax.numpy as jnp
from jax import lax
from jax.experimental import pallas as pl
from jax.experimental.pallas import tpu as pltpu
from jax.experimental.pallas import tpu_sc as plsc

N = 10000
E = 320000
B = 8
NPER = 1250
NH = 64
K = 625

NTILES = 32
EPT = E // NTILES
CH = 125
NCH = EPT // CH
ZROWS = 1000

_MESH = plsc.VectorSubcoreMesh(core_axis_name="c", subcore_axis_name="s")


_D = 4


def _sc_degree_body(c3, zeros_hbm, ones_hbm, out, cidx, ones_v, dsem, deg_sh):
    cid = lax.axis_index("c")
    sid = lax.axis_index("s")
    wid = cid * 16 + sid
    pltpu.sync_copy(c3.at[wid], cidx)
    pltpu.sync_copy(ones_hbm, ones_v)

    @pl.when(sid < N // ZROWS)
    def _():
        pltpu.sync_copy(zeros_hbm, deg_sh.at[pl.ds(sid * ZROWS, ZROWS)])

    plsc.subcore_barrier()

    def body(j, _):
        b = lax.rem(j, _D)

        @pl.when(j >= _D)
        def _():
            pltpu.make_async_copy(ones_v, deg_sh.at[cidx.at[j - _D]],
                                  dsem.at[b]).wait()

        pltpu.async_copy(ones_v, deg_sh.at[cidx.at[j]], dsem.at[b], add=True)
        return _

    lax.fori_loop(0, NCH, body, None)
    for j in range(NCH - _D, NCH):
        pltpu.make_async_copy(ones_v, deg_sh.at[cidx.at[j]],
                              dsem.at[j % _D]).wait()
    plsc.subcore_barrier()

    @pl.when(sid < N // ZROWS)
    def _():
        pltpu.sync_copy(deg_sh.at[pl.ds(sid * ZROWS, ZROWS)],
                        out.at[cid, pl.ds(sid * ZROWS, ZROWS)])


_W1 = 16

_sc_degree = pl.kernel(
    _sc_degree_body,
    out_type=jax.ShapeDtypeStruct((2, N, _W1), jnp.float32),
    mesh=_MESH,
    compiler_params=pltpu.CompilerParams(use_tc_tiling_on_sc=False),
    scratch_types=[
        pltpu.VMEM((NCH, CH), jnp.int32),
        pltpu.VMEM((CH, _W1), jnp.float32),
        pltpu.SemaphoreType.DMA((_D,)),
        pltpu.VMEM_SHARED((N, _W1), jnp.float32),
    ],
)


def _sc_scatter_body(u, r3, c3, zeros_hbm, out, ridx, cidx, rows, gsem, ssem,
                     s_sh):
    cid = lax.axis_index("c")
    sid = lax.axis_index("s")
    wid = cid * 16 + sid
    pltpu.sync_copy(r3.at[wid], ridx)
    pltpu.sync_copy(c3.at[wid], cidx)

    @pl.when(sid < N // ZROWS)
    def _():
        pltpu.sync_copy(zeros_hbm, s_sh.at[pl.ds(sid * ZROWS, ZROWS)])

    plsc.subcore_barrier()

    for j in range(_D):
        pltpu.async_copy(u.at[ridx.at[j]], rows.at[j], gsem.at[j])

    def body(j, _):
        b = lax.rem(j, _D)
        pltpu.make_async_copy(u.at[ridx.at[j]], rows.at[b], gsem.at[b]).wait()
        pltpu.async_copy(rows.at[b], s_sh.at[cidx.at[j]], ssem.at[b], add=True)

        @pl.when(j + _D < NCH)
        def _():
            pltpu.make_async_copy(rows.at[b], s_sh.at[cidx.at[j]],
                                  ssem.at[b]).wait()
            pltpu.async_copy(u.at[ridx.at[j + _D]], rows.at[b], gsem.at[b])
        return _

    lax.fori_loop(0, NCH, body, None)
    for j in range(NCH - _D, NCH):
        b = j % _D
        pltpu.make_async_copy(rows.at[b], s_sh.at[cidx.at[j]],
                              ssem.at[b]).wait()
    plsc.subcore_barrier()

    @pl.when(sid < N // ZROWS)
    def _():
        pltpu.sync_copy(s_sh.at[pl.ds(sid * ZROWS, ZROWS)],
                        out.at[cid, pl.ds(sid * ZROWS, ZROWS)])


def _make_sc_scatter(w):
    return pl.kernel(
        _sc_scatter_body,
        out_type=jax.ShapeDtypeStruct((2, N, w), jnp.float32),
        mesh=_MESH,
        compiler_params=pltpu.CompilerParams(use_tc_tiling_on_sc=False),
        scratch_types=[
            pltpu.VMEM((NCH, CH), jnp.int32),
            pltpu.VMEM((NCH, CH), jnp.int32),
            pltpu.VMEM((_D, CH, w), jnp.float32),
            pltpu.SemaphoreType.DMA((_D,)),
            pltpu.SemaphoreType.DMA((_D,)),
            pltpu.VMEM_SHARED((N, w), jnp.float32),
        ],
    )


_sc_scatter64 = _make_sc_scatter(NH)
_sc_scatter16 = _make_sc_scatter(_W1)


_R = 1000


def _tc1_body(degp, x, w1, dinv, u1):
    deg = degp[0] + degp[1] + 1.0
    di = lax.rsqrt(deg)
    dinv[...] = di
    u1[...] = jnp.dot(x[...], w1[...], preferred_element_type=jnp.float32) * di


def _tc_layer_body(sp, uprev, dinv, b, wnext, hprev, unext):
    h = jnp.maximum(dinv[...] * (sp[0] + sp[1] + uprev[...]) + b[...], 0.0)
    hprev[...] = h
    unext[...] = jnp.dot(h, wnext[...], preferred_element_type=jnp.float32) * dinv[...]


def _tc_score_body(sp, u3, dinv, b3, h1, h2, wsa, wsb, wsc, h3, us):
    h = jnp.maximum(dinv[...] * (sp[0] + sp[1] + u3[...]) + b3[...], 0.0)
    h3[...] = h
    s = (jnp.dot(h1[...], wsa[...], preferred_element_type=jnp.float32)
         + jnp.dot(h2[...], wsb[...], preferred_element_type=jnp.float32)
         + jnp.dot(h, wsc[...], preferred_element_type=jnp.float32))
    us[...] = jnp.broadcast_to(s * dinv[...], (s.shape[0], _W1))


def _full_spec(shape):
    return pl.BlockSpec(shape, lambda i: tuple(0 for _ in shape))


_tc1 = pl.pallas_call(
    _tc1_body,
    grid=(N // _R,),
    in_specs=[pl.BlockSpec((2, _R, 1), lambda i: (0, i, 0)),
              pl.BlockSpec((_R, 128), lambda i: (i, 0)),
              _full_spec((128, NH))],
    out_specs=[pl.BlockSpec((_R, 1), lambda i: (i, 0)),
               pl.BlockSpec((_R, NH), lambda i: (i, 0))],
    out_shape=[jax.ShapeDtypeStruct((N, 1), jnp.float32),
               jax.ShapeDtypeStruct((N, NH), jnp.float32)],
)

_tc_layer = pl.pallas_call(
    _tc_layer_body,
    grid=(N // _R,),
    in_specs=[pl.BlockSpec((2, _R, NH), lambda i: (0, i, 0)),
              pl.BlockSpec((_R, NH), lambda i: (i, 0)),
              pl.BlockSpec((_R, 1), lambda i: (i, 0)),
              _full_spec((1, NH)),
              _full_spec((NH, NH))],
    out_specs=[pl.BlockSpec((_R, NH), lambda i: (i, 0)),
               pl.BlockSpec((_R, NH), lambda i: (i, 0))],
    out_shape=[jax.ShapeDtypeStruct((N, NH), jnp.float32),
               jax.ShapeDtypeStruct((N, NH), jnp.float32)],
)

_tc_score = pl.pallas_call(
    _tc_score_body,
    grid=(N // _R,),
    in_specs=[pl.BlockSpec((2, _R, NH), lambda i: (0, i, 0)),
              pl.BlockSpec((_R, NH), lambda i: (i, 0)),
              pl.BlockSpec((_R, 1), lambda i: (i, 0)),
              _full_spec((1, NH)),
              pl.BlockSpec((_R, NH), lambda i: (i, 0)),
              pl.BlockSpec((_R, NH), lambda i: (i, 0)),
              _full_spec((NH, 1)),
              _full_spec((NH, 1)),
              _full_spec((NH, 1))],
    out_specs=[pl.BlockSpec((_R, NH), lambda i: (i, 0)),
               pl.BlockSpec((_R, _W1), lambda i: (i, 0))],
    out_shape=[jax.ShapeDtypeStruct((N, NH), jnp.float32),
               jax.ShapeDtypeStruct((N, _W1), jnp.float32)],
)


def _tc_mask_body(ssp, us, dinv, bs, gsel, negadd):
    score = dinv[...] * (ssp[0] + ssp[1] + us[...]) + bs[...]

    bits = lax.bitcast_convert_type(score, jnp.uint32)
    top = jnp.uint32(0x80000000)
    ukey = bits ^ jnp.where(bits >= top, jnp.uint32(0xFFFFFFFF), top)

    t = jnp.zeros((B, 1), jnp.uint32)
    for b in range(31, -1, -1):
        cand = t | jnp.uint32(1 << b)
        cnt = jnp.sum((ukey >= cand).astype(jnp.int32), axis=1, keepdims=True)
        t = jnp.where(cnt >= K, cand, t)

    gt = ukey > t
    tie = ukey == t
    r = K - jnp.sum(gt.astype(jnp.int32), axis=1, keepdims=True)
    idx = lax.broadcasted_iota(jnp.int32, (B, NPER), 1)
    lo = jnp.zeros((B, 1), jnp.int32)
    hi = jnp.full((B, 1), NPER - 1, jnp.int32)
    for _ in range(11):
        mid = (lo + hi) // 2
        cnt = jnp.sum((tie & (idx <= mid)).astype(jnp.int32), axis=1,
                      keepdims=True)
        ok = cnt >= r
        hi = jnp.where(ok, mid, hi)
        lo = jnp.where(ok, lo, mid + 1)
    mask = gt | (tie & (idx <= lo))

    gsel[...] = jnp.where(mask, jnp.tanh(score), 0.0)
    negadd[...] = jnp.where(mask, 0.0, jnp.float32(-3.4e38))


_tc_mask = pl.pallas_call(
    _tc_mask_body,
    out_shape=[jax.ShapeDtypeStruct((B, NPER), jnp.float32),
               jax.ShapeDtypeStruct((B, NPER), jnp.float32)],
)


def _tc_pool_body(h1, h2, h3, gsel, neg, wg, bg, wl2, bl2, wl3, bl3, out):
    g2 = gsel[0]
    n2 = neg[0]
    parts = []
    vs = [h[0] * g2 for h in (h1, h2, h3)]
    for v in vs:
        parts.append(jnp.max(v + n2, axis=0, keepdims=True))
    for v in vs:
        parts.append(jnp.sum(v, axis=0, keepdims=True) * (1.0 / K))
    xg = jnp.concatenate(parts, axis=1)
    y = jnp.maximum(jnp.dot(xg, wg[...], preferred_element_type=jnp.float32)
                    + bg[...], 0.0)
    y = jnp.maximum(jnp.dot(y, wl2[...], preferred_element_type=jnp.float32)
                    + bl2[...], 0.0)
    y = jnp.dot(y, wl3[...], preferred_element_type=jnp.float32) + bl3[...]
    z = y - jnp.max(y, axis=1, keepdims=True)
    out[0] = z - jnp.log(jnp.sum(jnp.exp(z), axis=1, keepdims=True))


_tc_pool = pl.pallas_call(
    _tc_pool_body,
    grid=(B,),
    in_specs=[pl.BlockSpec((1, NPER, NH), lambda g: (g, 0, 0)),
              pl.BlockSpec((1, NPER, NH), lambda g: (g, 0, 0)),
              pl.BlockSpec((1, NPER, NH), lambda g: (g, 0, 0)),
              pl.BlockSpec((1, NPER, 1), lambda g: (g, 0, 0)),
              pl.BlockSpec((1, NPER, 1), lambda g: (g, 0, 0)),
              _full_spec((6 * NH, NH)),
              _full_spec((1, NH)),
              _full_spec((NH, NH // 2)),
              _full_spec((1, NH // 2)),
              _full_spec((NH // 2, 10)),
              _full_spec((1, 10))],
    out_specs=pl.BlockSpec((1, 1, 10), lambda g: (g, 0, 0)),
    out_shape=jax.ShapeDtypeStruct((B, 1, 10), jnp.float32),
)


def kernel(x, edge_index, batch, batch_size, edge_attr, W1, b1, W2, b2, W3, b3,
           Ws, bs, Wg, bg, Wl2, bl2, Wl3, bl3):
    del batch, batch_size, edge_attr
    r3 = edge_index[0].astype(jnp.int32).reshape(NTILES, NCH, CH)
    c3 = edge_index[1].astype(jnp.int32).reshape(NTILES, NCH, CH)
    z64 = jnp.zeros((ZROWS, NH), jnp.float32)
    z16 = jnp.zeros((ZROWS, _W1), jnp.float32)
    ones16 = jnp.ones((CH, _W1), jnp.float32)

    degp = _sc_degree(c3, z16, ones16)
    dinv, u1 = _tc1(degp[:, :, 0:1], x, W1)

    s1 = _sc_scatter64(u1, r3, c3, z64)
    h1, u2 = _tc_layer(s1, u1, dinv, b1.reshape(1, NH), W2)

    s2 = _sc_scatter64(u2, r3, c3, z64)
    h2, u3 = _tc_layer(s2, u2, dinv, b2.reshape(1, NH), W3)

    s3 = _sc_scatter64(u3, r3, c3, z64)
    h3, us = _tc_score(s3, u3, dinv, b3.reshape(1, NH), h1, h2,
                       Ws[0:NH], Ws[NH:2 * NH], Ws[2 * NH:3 * NH])

    ss = _sc_scatter16(us, r3, c3, z16)

    gsel, negadd = _tc_mask(ss[:, :, 0].reshape(2, B, NPER),
                            us[:, 0].reshape(B, NPER),
                            dinv.reshape(B, NPER),
                            bs.reshape(1, 1))
    out = _tc_pool(
        h1.reshape(B, NPER, NH), h2.reshape(B, NPER, NH),
        h3.reshape(B, NPER, NH),
        gsel.reshape(B, NPER, 1), negadd.reshape(B, NPER, 1),
        Wg, bg.reshape(1, NH),
        Wl2, bl2.reshape(1, NH // 2),
        Wl3, bl3.reshape(1, 10))
    return out.reshape(B, 10)
